# baseline (speedup 1.0000x reference)
import functools

import jax
import jax.numpy as jnp
from jax.experimental import pallas as pl
from jax.experimental.pallas import tpu as pltpu

_BN = 512
_NCHUNK = 8


def _dot(x, w):
    return jax.lax.dot_general(
        x, w,
        dimension_numbers=(((1,), (1,)), ((), ())),
        precision=jax.lax.Precision.DEFAULT,
        preferred_element_type=jnp.float32,
    )


def _mm_kernel(x_hbm, w_hbm, b_ref, o_ref, xv_ref, w0_ref, w1_ref,
               xsems, wsems):
    m = xv_ref.shape[0]
    cm = m // _NCHUNK
    nsteps = pl.num_programs(0)
    j = pl.program_id(0)

    def x_copy(i):
        return pltpu.make_async_copy(
            x_hbm.at[pl.ds(i * cm, cm), :],
            xv_ref.at[pl.ds(i * cm, cm), :],
            xsems.at[i],
        )

    def w_copy(step, wbuf_ref, sem_i):
        return pltpu.make_async_copy(
            w_hbm.at[pl.ds(step * _BN, _BN), :],
            wbuf_ref,
            wsems.at[sem_i],
        )

    @pl.when(j == 0)
    def _first_step():
        w_copy(0, w0_ref, 0).start()
        x_copy(0).start()
        w_copy(1, w1_ref, 1).start()
        for i in range(1, _NCHUNK):
            x_copy(i).start()
        w_copy(0, w0_ref, 0).wait()
        for i in range(_NCHUNK):
            x_copy(i).wait()
            o_ref[pl.ds(i * cm, cm), :] = (
                _dot(xv_ref[pl.ds(i * cm, cm), :], w0_ref[...]) + b_ref[...])

    @pl.when(jnp.logical_and(j > 0, j % 2 == 1))
    def _odd_step():
        @pl.when(j + 1 < nsteps)
        def _prefetch():
            w_copy(j + 1, w0_ref, 0).start()
        w_copy(j, w1_ref, 1).wait()
        o_ref[...] = _dot(xv_ref[...], w1_ref[...]) + b_ref[...]

    @pl.when(jnp.logical_and(j > 0, j % 2 == 0))
    def _even_step():
        @pl.when(j + 1 < nsteps)
        def _prefetch():
            w_copy(j + 1, w1_ref, 1).start()
        w_copy(j, w0_ref, 0).wait()
        o_ref[...] = _dot(xv_ref[...], w0_ref[...]) + b_ref[...]


@functools.partial(jax.jit, static_argnames=())
def kernel(input, sparse_weight, bias):
    b, m, k = input.shape
    n = sparse_weight.shape[0]
    x = input.reshape(m, k)
    bias2 = bias.reshape(1, n)
    out = pl.pallas_call(
        _mm_kernel,
        grid=(n // _BN,),
        in_specs=[
            pl.BlockSpec(memory_space=pl.ANY),
            pl.BlockSpec(memory_space=pl.ANY),
            pl.BlockSpec((1, _BN), lambda j: (0, j)),
        ],
        out_specs=pl.BlockSpec((m, _BN), lambda j: (0, j)),
        out_shape=jax.ShapeDtypeStruct((m, n), jnp.float32),
        scratch_shapes=[
            pltpu.VMEM((m, k), jnp.float32),
            pltpu.VMEM((_BN, k), jnp.float32),
            pltpu.VMEM((_BN, k), jnp.float32),
            pltpu.SemaphoreType.DMA((_NCHUNK,)),
            pltpu.SemaphoreType.DMA((2,)),
        ],
        compiler_params=pltpu.CompilerParams(
            vmem_limit_bytes=62 * 1024 * 1024),
    )(x, sparse_weight, bias2)
    return out.reshape(b, m, n)


# speedup vs baseline: 1.0548x; 1.0548x over previous
import functools

import jax
import jax.numpy as jnp
from jax.experimental import pallas as pl
from jax.experimental.pallas import tpu as pltpu

_BN = 512
_NCHUNK = 8


def _dot(x, w, b):
    acc = jax.lax.dot_general(
        x, w,
        dimension_numbers=(((1,), (1,)), ((), ())),
        precision=jax.lax.Precision.DEFAULT,
        preferred_element_type=jnp.float32,
    )
    return acc + b


def _mm_kernel(x_hbm, w_ref, b_ref, o_ref, xv_ref, sems):
    m = xv_ref.shape[0]
    cm = m // _NCHUNK

    @pl.when(pl.program_id(0) == 0)
    def _first_step():
        for i in range(_NCHUNK):
            pltpu.make_async_copy(
                x_hbm.at[pl.ds(i * cm, cm), :],
                xv_ref.at[pl.ds(i * cm, cm), :],
                sems.at[i],
            ).start()
        for i in range(_NCHUNK):
            pltpu.make_async_copy(
                x_hbm.at[pl.ds(i * cm, cm), :],
                xv_ref.at[pl.ds(i * cm, cm), :],
                sems.at[i],
            ).wait()
            o_ref[pl.ds(i * cm, cm), :] = _dot(
                xv_ref[pl.ds(i * cm, cm), :], w_ref[...], b_ref[...])

    @pl.when(pl.program_id(0) != 0)
    def _rest():
        o_ref[...] = _dot(xv_ref[...], w_ref[...], b_ref[...])


@functools.partial(jax.jit, static_argnames=())
def kernel(input, sparse_weight, bias):
    b, m, k = input.shape
    n = sparse_weight.shape[0]
    x = input.reshape(m, k)
    bias2 = bias.reshape(1, n)
    out = pl.pallas_call(
        _mm_kernel,
        grid=(n // _BN,),
        in_specs=[
            pl.BlockSpec(memory_space=pl.ANY),
            pl.BlockSpec((_BN, k), lambda j: (j, 0)),
            pl.BlockSpec((1, _BN), lambda j: (0, j)),
        ],
        out_specs=pl.BlockSpec((m, _BN), lambda j: (0, j)),
        out_shape=jax.ShapeDtypeStruct((m, n), jnp.float32),
        scratch_shapes=[
            pltpu.VMEM((m, k), jnp.float32),
            pltpu.SemaphoreType.DMA((_NCHUNK,)),
        ],
        compiler_params=pltpu.CompilerParams(
            vmem_limit_bytes=62 * 1024 * 1024),
    )(x, sparse_weight, bias2)
    return out.reshape(b, m, n)
